# baseline (speedup 1.0000x reference)
import functools

import jax
import jax.numpy as jnp
from jax import lax
from jax.experimental import pallas as pl
from jax.experimental.pallas import tpu as pltpu
from jax.experimental.pallas import tpu_sc as plsc

N = 10000
E = 320000
H = 128
EPS = 1e-07

NC = 2
NS = 16
K = 80
KH = K // 2
NCHUNKS = E // K
NCH_T = NCHUNKS // NS
SUP = 5
NSUP_T = NCH_T // SUP
NP = 10240
ROWS_T = NP // NS

HW = H // 2


def _pack_bf16_words(zlo, zhi):
    ulo = lax.bitcast_convert_type(zlo, jnp.uint32)
    uhi = lax.bitcast_convert_type(zhi, jnp.uint32)
    ulo = ulo + jnp.uint32(0x7FFF) + ((ulo >> 16) & jnp.uint32(1))
    uhi = uhi + jnp.uint32(0x7FFF) + ((uhi >> 16) & jnp.uint32(1))
    word = (uhi & jnp.uint32(0xFFFF0000)) | (ulo >> 16)
    return lax.bitcast_convert_type(word, jnp.int32)


def _unpack_bf16_words_tc(word):
    u = lax.bitcast_convert_type(word, jnp.uint32)
    lo = lax.bitcast_convert_type(u << 16, jnp.float32)
    hi = lax.bitcast_convert_type(u & jnp.uint32(0xFFFF0000), jnp.float32)
    return lo, hi


def _node_enc_body(x_ref, w_ref, b_ref, h_ref, hmax_ref):
    h = jnp.dot(x_ref[...], w_ref[...], preferred_element_type=jnp.float32)
    h = h + b_ref[...]
    h_ref[...] = h
    hmax_ref[...] = jnp.max(h, axis=0, keepdims=True)


def _node_enc(xp, wp, b):
    return pl.pallas_call(
        _node_enc_body,
        out_shape=(
            jax.ShapeDtypeStruct((N, H), jnp.float32),
            jax.ShapeDtypeStruct((1, H), jnp.float32),
        ),
    )(xp, wp, b)


def _edge_enc_body(ea_ref, wlo_ref, blo_ref, whi_ref, bhi_ref,
                   emb_ref, emax_ref):
    i = pl.program_id(0)
    zlo = jnp.dot(ea_ref[...], wlo_ref[...],
                  preferred_element_type=jnp.float32) + blo_ref[...]
    zhi = jnp.dot(ea_ref[...], whi_ref[...],
                  preferred_element_type=jnp.float32) + bhi_ref[...]
    word = _pack_bf16_words(zlo, zhi)
    emb_ref[...] = word
    rlo, rhi = _unpack_bf16_words_tc(word)
    m = jnp.concatenate(
        [jnp.max(rlo, axis=0, keepdims=True),
         jnp.max(rhi, axis=0, keepdims=True)], axis=1)

    @pl.when(i == 0)
    def _():
        emax_ref[...] = m

    @pl.when(i > 0)
    def _():
        emax_ref[...] = jnp.maximum(emax_ref[...], m)


def _edge_enc(eap, wp, b, eblk=8000):
    grid = (E // eblk,)
    return pl.pallas_call(
        _edge_enc_body,
        grid=grid,
        in_specs=[
            pl.BlockSpec((eblk, 8), lambda i: (i, 0)),
            pl.BlockSpec((8, HW), lambda i: (0, 0)),
            pl.BlockSpec((1, HW), lambda i: (0, 0)),
            pl.BlockSpec((8, HW), lambda i: (0, 0)),
            pl.BlockSpec((1, HW), lambda i: (0, 0)),
        ],
        out_specs=(
            pl.BlockSpec((eblk, HW), lambda i: (i, 0)),
            pl.BlockSpec((1, H), lambda i: (0, 0)),
        ),
        out_shape=(
            jax.ShapeDtypeStruct((E, HW), jnp.int32),
            jax.ShapeDtypeStruct((1, H), jnp.float32),
        ),
    )(eap, wp[:, :HW], b[:, :HW], wp[:, HW:], b[:, HW:])


def _post_mid_body(h_ref, s_ref, u_ref, w_ref, b_ref, g_ref, be_ref,
                   out_ref, omax_ref):
    x = h_ref[...] + u_ref[...] / (s_ref[...] + 1e-16)
    z = jnp.dot(x, w_ref[...], preferred_element_type=jnp.float32) + b_ref[...]
    mu = jnp.mean(z, axis=-1, keepdims=True)
    zc = z - mu
    var = jnp.mean(zc * zc, axis=-1, keepdims=True)
    inv = lax.rsqrt(var + 1e-05)
    y = jnp.maximum(zc * inv * g_ref[...] + be_ref[...], 0.0)
    out_ref[...] = y
    omax_ref[...] = jnp.max(y, axis=0, keepdims=True)


def _post_mid(h, s, u, w, b, g, be):
    return pl.pallas_call(
        _post_mid_body,
        out_shape=(
            jax.ShapeDtypeStruct((N, H), jnp.float32),
            jax.ShapeDtypeStruct((1, H), jnp.float32),
        ),
    )(h, s, u, w, b, g, be)


def _post_final_body(h_ref, s_ref, u_ref, w_ref, b_ref, g_ref, be_ref,
                     out_ref):
    x = h_ref[...] + u_ref[...] / (s_ref[...] + 1e-16)
    z = jnp.dot(x, w_ref[...], preferred_element_type=jnp.float32) + b_ref[...]
    mu = jnp.mean(z, axis=-1, keepdims=True)
    zc = z - mu
    var = jnp.mean(zc * zc, axis=-1, keepdims=True)
    out_ref[...] = zc * lax.rsqrt(var + 1e-05) * g_ref[...] + be_ref[...]


def _post_final(h, s, u, w, b, g, be):
    return pl.pallas_call(
        _post_final_body,
        out_shape=jax.ShapeDtypeStruct((N, H), jnp.float32),
    )(h, s, u, w, b, g, be)


def _sc_aggr_body(h_hbm, emb_hbm, src_hbm, dst_hbm, bnd_hbm,
                  s_out, u_out,
                  src_v, dst_v, rows_v, emb_v, out_v, bnd_v,
                  accum, sem_emb, sem_rows, sem_sc, sem_idx):
    c = lax.axis_index("c")
    t = lax.axis_index("s")
    e_base = t * NCH_T * K

    pltpu.sync_copy(bnd_hbm, bnd_v)

    def zrow(r, carry):
        for v in range(8):
            out_v[r, pl.ds(v * 16, 16)] = jnp.zeros((16,), jnp.float32)
        return carry

    lax.fori_loop(0, K, zrow, 0)
    for z in range(ROWS_T // K):
        pltpu.sync_copy(out_v, accum.at[pl.ds(t * ROWS_T + z * K, K)])
    plsc.subcore_barrier()

    def emb_slice(ci, p):
        return emb_hbm.at[pl.ds(e_base + ci * K + p * KH, KH)]

    def src_row(ci):
        return src_v.at[lax.rem(lax.div(ci, SUP), 3), lax.rem(ci, SUP)]

    def dst_row(ci):
        return dst_v.at[lax.rem(lax.div(ci, SUP), 3), lax.rem(ci, SUP)]

    def load_super(sj):
        slot = lax.rem(sj, 3)
        pltpu.async_copy(src_hbm.at[t * NSUP_T + sj], src_v.at[slot], sem_idx)
        pltpu.async_copy(dst_hbm.at[t * NSUP_T + sj], dst_v.at[slot], sem_idx)

    def wait_super():
        pltpu.make_async_copy(src_hbm.at[t * NSUP_T], src_v.at[0],
                              sem_idx).wait()
        pltpu.make_async_copy(dst_hbm.at[t * NSUP_T], dst_v.at[0],
                              sem_idx).wait()

    load_super(0)
    wait_super()
    load_super(1)
    pltpu.async_copy(h_hbm.at[src_row(0)], rows_v.at[0], sem_rows)
    pltpu.async_copy(emb_slice(0, 0), emb_v.at[0], sem_emb)

    bnd8 = tuple(bnd_v[0, pl.ds(v * 16, 16)] for v in range(8))
    himask = jnp.full((16,), -65536, jnp.int32)

    def compute_half(b, p):

        def msg(r, g):
            hlo = rows_v[b, KH * p + r, pl.ds(16 * g, 16)]
            hhi = rows_v[b, KH * p + r, pl.ds(64 + 16 * g, 16)]
            we = emb_v[p, r, pl.ds(16 * g, 16)]
            elo = lax.bitcast_convert_type(lax.shift_left(we, 16),
                                           jnp.float32)
            ehi = lax.bitcast_convert_type(we & himask, jnp.float32)
            m1 = jnp.maximum(hlo + elo, 0.0) + EPS
            m2 = jnp.maximum(hhi + ehi, 0.0) + EPS
            return m1, m2

        @pl.when(c == 0)
        def _():
            def row(r, carry):
                for g in range(4):
                    m1, m2 = msg(r, g)
                    out_v[KH * p + r, pl.ds(16 * g, 16)] = (
                        jnp.exp(m1 - carry[g]))
                    out_v[KH * p + r, pl.ds(64 + 16 * g, 16)] = (
                        jnp.exp(m2 - carry[4 + g]))
                return carry

            lax.fori_loop(0, KH, row, bnd8)

        @pl.when(c == 1)
        def _():
            def row(r, carry):
                for g in range(4):
                    m1, m2 = msg(r, g)
                    out_v[KH * p + r, pl.ds(16 * g, 16)] = (
                        jnp.exp(m1 - carry[g]) * m1)
                    out_v[KH * p + r, pl.ds(64 + 16 * g, 16)] = (
                        jnp.exp(m2 - carry[4 + g]) * m2)
                return carry

            lax.fori_loop(0, KH, row, bnd8)

    def macro(ci, carry):
        b = lax.rem(ci, 2)
        @pl.when((lax.rem(ci + 1, SUP) == 0) & (ci + 1 < NCH_T))
        def _():
            wait_super()

            @pl.when(lax.div(ci + 1, SUP) + 1 < NSUP_T)
            def _():
                load_super(lax.div(ci + 1, SUP) + 1)

        pltpu.make_async_copy(h_hbm.at[src_v.at[0, 0]], rows_v.at[0],
                              sem_rows).wait()

        @pl.when(ci + 1 < NCH_T)
        def _():
            pltpu.async_copy(h_hbm.at[src_row(ci + 1)], rows_v.at[1 - b],
                             sem_rows)

        @pl.when(ci >= 1)
        def _():
            pltpu.make_async_copy(out_v, accum.at[dst_v.at[0, 0]],
                                  sem_sc).wait()

        for p in range(2):
            pltpu.make_async_copy(emb_slice(0, 0), emb_v.at[p],
                                  sem_emb).wait()
            if p == 0:
                pltpu.async_copy(emb_slice(ci, 1), emb_v.at[1], sem_emb)
            else:
                @pl.when(ci + 1 < NCH_T)
                def _():
                    pltpu.async_copy(emb_slice(ci + 1, 0), emb_v.at[0],
                                     sem_emb)
            compute_half(b, p)

        pltpu.async_copy(out_v, accum.at[dst_row(ci)], sem_sc, add=True)
        return carry

    lax.fori_loop(0, NCH_T, macro, 0)
    pltpu.make_async_copy(out_v, accum.at[dst_v.at[0, 0]], sem_sc).wait()
    plsc.subcore_barrier()

    @pl.when(c == 0)
    def _():
        pltpu.sync_copy(accum.at[pl.ds(t * ROWS_T, ROWS_T)],
                        s_out.at[pl.ds(t * ROWS_T, ROWS_T)])

    @pl.when(c == 1)
    def _():
        pltpu.sync_copy(accum.at[pl.ds(t * ROWS_T, ROWS_T)],
                        u_out.at[pl.ds(t * ROWS_T, ROWS_T)])


_sc_aggr = functools.partial(
    pl.kernel,
    out_type=(
        jax.ShapeDtypeStruct((NP, H), jnp.float32),
        jax.ShapeDtypeStruct((NP, H), jnp.float32),
    ),
    mesh=plsc.VectorSubcoreMesh(core_axis_name="c", subcore_axis_name="s"),
    scratch_types=[
        pltpu.VMEM((3, SUP, K), jnp.int32),
        pltpu.VMEM((3, SUP, K), jnp.int32),
        pltpu.VMEM((2, K, H), jnp.float32),
        pltpu.VMEM((2, KH, HW), jnp.int32),
        pltpu.VMEM((K, H), jnp.float32),
        pltpu.VMEM((1, H), jnp.float32),
        pltpu.VMEM_SHARED((NP, H), jnp.float32),
        pltpu.SemaphoreType.DMA,
        pltpu.SemaphoreType.DMA,
        pltpu.SemaphoreType.DMA,
        pltpu.SemaphoreType.DMA,
    ],
)(_sc_aggr_body)


def kernel(x, edge_index, edge_attr, W_node, b_node, W_edge, b_edge,
           Wc0, bc0, Wc1, bc1, g0, be0, g1, be1):
    xp = jnp.pad(x, ((0, 0), (0, 5)))
    Wn = jnp.pad(W_node, ((0, 5), (0, 0)))
    eap = jnp.pad(edge_attr, ((0, 0), (0, 1)))
    We = jnp.pad(W_edge, ((0, 1), (0, 0)))

    h, hmax = _node_enc(xp, Wn, b_node.reshape(1, H))
    emb, emax = _edge_enc(eap, We, b_edge.reshape(1, H))

    src3d = edge_index[0].reshape(NS * NSUP_T, SUP, K)
    dst3d = edge_index[1].reshape(NS * NSUP_T, SUP, K)

    bnd0 = jnp.maximum(hmax + emax, 0.0) + EPS
    s0, u0 = _sc_aggr(h, emb, src3d, dst3d, bnd0)
    h2, h2max = _post_mid(h, s0[:N], u0[:N], Wc0, bc0.reshape(1, H),
                          g0.reshape(1, H), be0.reshape(1, H))

    bnd1 = jnp.maximum(h2max + emax, 0.0) + EPS
    s1, u1 = _sc_aggr(h2, emb, src3d, dst3d, bnd1)
    return _post_final(h2, s1[:N], u1[:N], Wc1, bc1.reshape(1, H),
                       g1.reshape(1, H), be1.reshape(1, H))


# speedup vs baseline: 2.4478x; 2.4478x over previous
import functools

import jax
import jax.numpy as jnp
from jax import lax
from jax.experimental import pallas as pl
from jax.experimental.pallas import tpu as pltpu
from jax.experimental.pallas import tpu_sc as plsc

N = 10000
E = 320000
H = 128
EPS = 1e-07

NC = 2
NS = 16
K = 80
KH = K // 2
NCHUNKS = E // K
NCH_T = NCHUNKS // NS
SUP = 5
NSUP_T = NCH_T // SUP
NP = 10240
ROWS_T = NP // NS

HW = H // 2


def _pack_bf16_words(zlo, zhi):
    ulo = lax.bitcast_convert_type(zlo, jnp.uint32)
    uhi = lax.bitcast_convert_type(zhi, jnp.uint32)
    ulo = ulo + jnp.uint32(0x7FFF) + ((ulo >> 16) & jnp.uint32(1))
    uhi = uhi + jnp.uint32(0x7FFF) + ((uhi >> 16) & jnp.uint32(1))
    word = (uhi & jnp.uint32(0xFFFF0000)) | (ulo >> 16)
    return lax.bitcast_convert_type(word, jnp.int32)


def _unpack_bf16_words_tc(word):
    u = lax.bitcast_convert_type(word, jnp.uint32)
    lo = lax.bitcast_convert_type(u << 16, jnp.float32)
    hi = lax.bitcast_convert_type(u & jnp.uint32(0xFFFF0000), jnp.float32)
    return lo, hi


def _node_enc_body(x_ref, w_ref, b_ref, h_ref, hmax_ref):
    h = jnp.dot(x_ref[...], w_ref[...], preferred_element_type=jnp.float32)
    h = h + b_ref[...]
    h_ref[...] = h
    hmax_ref[...] = jnp.max(h, axis=0, keepdims=True)


def _node_enc(xp, wp, b):
    return pl.pallas_call(
        _node_enc_body,
        out_shape=(
            jax.ShapeDtypeStruct((N, H), jnp.float32),
            jax.ShapeDtypeStruct((1, H), jnp.float32),
        ),
    )(xp, wp, b)


def _edge_enc_body(ea_ref, wlo_ref, blo_ref, whi_ref, bhi_ref,
                   emb_ref, emax_ref):
    i = pl.program_id(0)
    zlo = jnp.dot(ea_ref[...], wlo_ref[...],
                  preferred_element_type=jnp.float32) + blo_ref[...]
    zhi = jnp.dot(ea_ref[...], whi_ref[...],
                  preferred_element_type=jnp.float32) + bhi_ref[...]
    word = _pack_bf16_words(zlo, zhi)
    emb_ref[...] = word
    rlo, rhi = _unpack_bf16_words_tc(word)
    m = jnp.concatenate(
        [jnp.max(rlo, axis=0, keepdims=True),
         jnp.max(rhi, axis=0, keepdims=True)], axis=1)

    @pl.when(i == 0)
    def _():
        emax_ref[...] = m

    @pl.when(i > 0)
    def _():
        emax_ref[...] = jnp.maximum(emax_ref[...], m)


def _edge_enc(eap, wp, b, eblk=8000):
    grid = (E // eblk,)
    return pl.pallas_call(
        _edge_enc_body,
        grid=grid,
        in_specs=[
            pl.BlockSpec((eblk, 8), lambda i: (i, 0)),
            pl.BlockSpec((8, HW), lambda i: (0, 0)),
            pl.BlockSpec((1, HW), lambda i: (0, 0)),
            pl.BlockSpec((8, HW), lambda i: (0, 0)),
            pl.BlockSpec((1, HW), lambda i: (0, 0)),
        ],
        out_specs=(
            pl.BlockSpec((eblk, HW), lambda i: (i, 0)),
            pl.BlockSpec((1, H), lambda i: (0, 0)),
        ),
        out_shape=(
            jax.ShapeDtypeStruct((E, HW), jnp.int32),
            jax.ShapeDtypeStruct((1, H), jnp.float32),
        ),
    )(eap, wp[:, :HW], b[:, :HW], wp[:, HW:], b[:, HW:])


def _post_mid_body(h_ref, s_ref, u_ref, w_ref, b_ref, g_ref, be_ref,
                   out_ref, omax_ref):
    x = h_ref[...] + u_ref[...] / (s_ref[...] + 1e-16)
    z = jnp.dot(x, w_ref[...], preferred_element_type=jnp.float32) + b_ref[...]
    mu = jnp.mean(z, axis=-1, keepdims=True)
    zc = z - mu
    var = jnp.mean(zc * zc, axis=-1, keepdims=True)
    inv = lax.rsqrt(var + 1e-05)
    y = jnp.maximum(zc * inv * g_ref[...] + be_ref[...], 0.0)
    out_ref[...] = y
    omax_ref[...] = jnp.max(y, axis=0, keepdims=True)


def _post_mid(h, s, u, w, b, g, be):
    return pl.pallas_call(
        _post_mid_body,
        out_shape=(
            jax.ShapeDtypeStruct((N, H), jnp.float32),
            jax.ShapeDtypeStruct((1, H), jnp.float32),
        ),
    )(h, s, u, w, b, g, be)


def _post_final_body(h_ref, s_ref, u_ref, w_ref, b_ref, g_ref, be_ref,
                     out_ref):
    x = h_ref[...] + u_ref[...] / (s_ref[...] + 1e-16)
    z = jnp.dot(x, w_ref[...], preferred_element_type=jnp.float32) + b_ref[...]
    mu = jnp.mean(z, axis=-1, keepdims=True)
    zc = z - mu
    var = jnp.mean(zc * zc, axis=-1, keepdims=True)
    out_ref[...] = zc * lax.rsqrt(var + 1e-05) * g_ref[...] + be_ref[...]


def _post_final(h, s, u, w, b, g, be):
    return pl.pallas_call(
        _post_final_body,
        out_shape=jax.ShapeDtypeStruct((N, H), jnp.float32),
    )(h, s, u, w, b, g, be)


def _sc_aggr_body(h_hbm, emb_hbm, src_hbm, dst_hbm, bnd_hbm,
                  s_out, u_out,
                  src_v, dst_v, rows_v, emb_v, out_v, bnd_v,
                  accum, sem_emb, sem_rows, sem_sc, sem_idx):
    c = lax.axis_index("c")
    t = lax.axis_index("s")
    e_base = t * NCH_T * K

    pltpu.sync_copy(bnd_hbm, bnd_v)

    def zrow(r, carry):
        for v in range(8):
            out_v[r, pl.ds(v * 16, 16)] = jnp.zeros((16,), jnp.float32)
        return carry

    lax.fori_loop(0, K, zrow, 0)
    for z in range(ROWS_T // K):
        pltpu.sync_copy(out_v, accum.at[pl.ds(t * ROWS_T + z * K, K)])
    plsc.subcore_barrier()

    def emb_slice(ci, p):
        return emb_hbm.at[pl.ds(e_base + ci * K + p * KH, KH)]

    def src_row(ci):
        return src_v.at[lax.rem(lax.div(ci, SUP), 2), lax.rem(ci, SUP)]

    def dst_row_half(ci, p):
        return dst_v.at[lax.rem(lax.div(ci, SUP), 2), 2 * lax.rem(ci, SUP) + p]

    def load_super(sj):
        slot = lax.rem(sj, 2)
        pltpu.async_copy(src_hbm.at[t * NSUP_T + sj], src_v.at[slot], sem_idx)
        pltpu.async_copy(dst_hbm.at[t * NSUP_T + sj], dst_v.at[slot], sem_idx)

    def wait_super():
        pltpu.make_async_copy(src_hbm.at[t * NSUP_T], src_v.at[0],
                              sem_idx).wait()
        pltpu.make_async_copy(dst_hbm.at[t * NSUP_T], dst_v.at[0],
                              sem_idx).wait()

    load_super(0)
    wait_super()
    load_super(1)
    pltpu.async_copy(h_hbm.at[src_row(0)], rows_v.at[0], sem_rows)
    pltpu.async_copy(emb_slice(0, 0), emb_v.at[0], sem_emb)

    bnd8 = tuple(bnd_v[0, pl.ds(v * 16, 16)] for v in range(8))
    himask = jnp.full((16,), -65536, jnp.int32)

    def compute_half(b, p):

        def msg(r, g):
            hlo = rows_v[b, KH * p + r, pl.ds(16 * g, 16)]
            hhi = rows_v[b, KH * p + r, pl.ds(64 + 16 * g, 16)]
            we = emb_v[p, r, pl.ds(16 * g, 16)]
            elo = lax.bitcast_convert_type(lax.shift_left(we, 16),
                                           jnp.float32)
            ehi = lax.bitcast_convert_type(we & himask, jnp.float32)
            m1 = jnp.maximum(hlo + elo, 0.0) + EPS
            m2 = jnp.maximum(hhi + ehi, 0.0) + EPS
            return m1, m2

        @pl.when(c == 0)
        def _():
            def row(r, carry):
                for g in range(4):
                    m1, m2 = msg(r, g)
                    out_v[KH * p + r, pl.ds(16 * g, 16)] = (
                        jnp.exp(m1 - carry[g]))
                    out_v[KH * p + r, pl.ds(64 + 16 * g, 16)] = (
                        jnp.exp(m2 - carry[4 + g]))
                return carry

            lax.fori_loop(0, KH, row, bnd8)

        @pl.when(c == 1)
        def _():
            def row(r, carry):
                for g in range(4):
                    m1, m2 = msg(r, g)
                    out_v[KH * p + r, pl.ds(16 * g, 16)] = (
                        jnp.exp(m1 - carry[g]) * m1)
                    out_v[KH * p + r, pl.ds(64 + 16 * g, 16)] = (
                        jnp.exp(m2 - carry[4 + g]) * m2)
                return carry

            lax.fori_loop(0, KH, row, bnd8)

    def macro(ci, b):
        pltpu.make_async_copy(h_hbm.at[src_v.at[0, 0]], rows_v.at[b],
                              sem_rows).wait()

        @pl.when((lax.rem(ci + 1, SUP) == 0) & (ci + 1 < NCH_T))
        def _():
            for _ in range(2):
                pltpu.make_async_copy(out_v.at[pl.ds(0, KH)],
                                      accum.at[dst_v.at[0, 0]],
                                      sem_sc).wait()
            wait_super()

            @pl.when(lax.div(ci + 1, SUP) + 1 < NSUP_T)
            def _():
                load_super(lax.div(ci + 1, SUP) + 1)

        @pl.when(ci + 1 < NCH_T)
        def _():
            pltpu.async_copy(h_hbm.at[src_row(ci + 1)], rows_v.at[1 - b],
                             sem_rows)

        for p in range(2):
            pltpu.make_async_copy(emb_slice(0, 0), emb_v.at[p],
                                  sem_emb).wait()
            if p == 0:
                pltpu.async_copy(emb_slice(ci, 1), emb_v.at[1], sem_emb)
            else:
                @pl.when(ci + 1 < NCH_T)
                def _():
                    pltpu.async_copy(emb_slice(ci + 1, 0), emb_v.at[0],
                                     sem_emb)

            @pl.when((ci >= 1) & ((lax.rem(ci + 1, SUP) != 0)
                                  | (ci + 1 >= NCH_T)))
            def _():
                pltpu.make_async_copy(out_v.at[pl.ds(0, KH)],
                                      accum.at[dst_v.at[0, 0]],
                                      sem_sc).wait()

            compute_half(b, p)
            pltpu.async_copy(out_v.at[pl.ds(KH * p, KH)],
                             accum.at[dst_row_half(ci, p)], sem_sc, add=True)

    def pair(g, carry):
        for b in range(2):
            macro(2 * g + b, b)
        return carry

    lax.fori_loop(0, NCH_T // 2, pair, 0)
    for p in range(2):
        pltpu.make_async_copy(out_v.at[pl.ds(0, KH)],
                              accum.at[dst_v.at[0, 0]], sem_sc).wait()
    plsc.subcore_barrier()

    @pl.when(c == 0)
    def _():
        pltpu.sync_copy(accum.at[pl.ds(t * ROWS_T, ROWS_T)],
                        s_out.at[pl.ds(t * ROWS_T, ROWS_T)])

    @pl.when(c == 1)
    def _():
        pltpu.sync_copy(accum.at[pl.ds(t * ROWS_T, ROWS_T)],
                        u_out.at[pl.ds(t * ROWS_T, ROWS_T)])


_sc_aggr = functools.partial(
    pl.kernel,
    out_type=(
        jax.ShapeDtypeStruct((NP, H), jnp.float32),
        jax.ShapeDtypeStruct((NP, H), jnp.float32),
    ),
    mesh=plsc.VectorSubcoreMesh(core_axis_name="c", subcore_axis_name="s"),
    scratch_types=[
        pltpu.VMEM((2, SUP, K), jnp.int32),
        pltpu.VMEM((2, 2 * SUP, KH), jnp.int32),
        pltpu.VMEM((2, K, H), jnp.float32),
        pltpu.VMEM((2, KH, HW), jnp.int32),
        pltpu.VMEM((K, H), jnp.float32),
        pltpu.VMEM((1, H), jnp.float32),
        pltpu.VMEM_SHARED((NP, H), jnp.float32),
        pltpu.SemaphoreType.DMA,
        pltpu.SemaphoreType.DMA,
        pltpu.SemaphoreType.DMA,
        pltpu.SemaphoreType.DMA,
    ],
)(_sc_aggr_body)


def kernel(x, edge_index, edge_attr, W_node, b_node, W_edge, b_edge,
           Wc0, bc0, Wc1, bc1, g0, be0, g1, be1):
    xp = jnp.pad(x, ((0, 0), (0, 5)))
    Wn = jnp.pad(W_node, ((0, 5), (0, 0)))
    eap = jnp.pad(edge_attr, ((0, 0), (0, 1)))
    We = jnp.pad(W_edge, ((0, 1), (0, 0)))

    h, hmax = _node_enc(xp, Wn, b_node.reshape(1, H))
    emb, emax = _edge_enc(eap, We, b_edge.reshape(1, H))

    src3d = edge_index[0].reshape(NS * NSUP_T, SUP, K)
    dst3d = edge_index[1].reshape(NS * NSUP_T, 2 * SUP, KH)

    bnd0 = jnp.maximum(hmax + emax, 0.0) + EPS
    s0, u0 = _sc_aggr(h, emb, src3d, dst3d, bnd0)
    h2, h2max = _post_mid(h, s0[:N], u0[:N], Wc0, bc0.reshape(1, H),
                          g0.reshape(1, H), be0.reshape(1, H))

    bnd1 = jnp.maximum(h2max + emax, 0.0) + EPS
    s1, u1 = _sc_aggr(h2, emb, src3d, dst3d, bnd1)
    return _post_final(h2, s1[:N], u1[:N], Wc1, bc1.reshape(1, H),
                       g1.reshape(1, H), be1.reshape(1, H))


# speedup vs baseline: 2.6955x; 1.1012x over previous
import functools

import jax
import jax.numpy as jnp
from jax import lax
from jax.experimental import pallas as pl
from jax.experimental.pallas import tpu as pltpu
from jax.experimental.pallas import tpu_sc as plsc

N = 10000
E = 320000
H = 128
EPS = 1e-07

NC = 2
NS = 16
K = 40
NCHUNKS = E // K
NCH_T = NCHUNKS // NS
SUP = 10
NSUP_T = NCH_T // SUP
NP = 10240
ROWS_T = NP // NS

HW = H // 2


def _pack_bf16_words(zlo, zhi):
    ulo = lax.bitcast_convert_type(zlo, jnp.uint32)
    uhi = lax.bitcast_convert_type(zhi, jnp.uint32)
    ulo = ulo + jnp.uint32(0x7FFF) + ((ulo >> 16) & jnp.uint32(1))
    uhi = uhi + jnp.uint32(0x7FFF) + ((uhi >> 16) & jnp.uint32(1))
    word = (uhi & jnp.uint32(0xFFFF0000)) | (ulo >> 16)
    return lax.bitcast_convert_type(word, jnp.int32)


def _unpack_bf16_words_tc(word):
    u = lax.bitcast_convert_type(word, jnp.uint32)
    lo = lax.bitcast_convert_type(u << 16, jnp.float32)
    hi = lax.bitcast_convert_type(u & jnp.uint32(0xFFFF0000), jnp.float32)
    return lo, hi


def _node_enc_body(x_ref, w_ref, b_ref, h_ref, hmax_ref):
    h = jnp.dot(x_ref[...], w_ref[...], preferred_element_type=jnp.float32)
    h = h + b_ref[...]
    h_ref[...] = h
    hmax_ref[...] = jnp.max(h, axis=0, keepdims=True)


def _node_enc(xp, wp, b):
    return pl.pallas_call(
        _node_enc_body,
        out_shape=(
            jax.ShapeDtypeStruct((N, H), jnp.float32),
            jax.ShapeDtypeStruct((1, H), jnp.float32),
        ),
    )(xp, wp, b)


def _edge_enc_body(ea_ref, wlo_ref, blo_ref, whi_ref, bhi_ref,
                   emb_ref, emax_ref):
    i = pl.program_id(0)
    zlo = jnp.dot(ea_ref[...], wlo_ref[...],
                  preferred_element_type=jnp.float32) + blo_ref[...]
    zhi = jnp.dot(ea_ref[...], whi_ref[...],
                  preferred_element_type=jnp.float32) + bhi_ref[...]
    word = _pack_bf16_words(zlo, zhi)
    emb_ref[...] = word
    rlo, rhi = _unpack_bf16_words_tc(word)
    m = jnp.concatenate(
        [jnp.max(rlo, axis=0, keepdims=True),
         jnp.max(rhi, axis=0, keepdims=True)], axis=1)

    @pl.when(i == 0)
    def _():
        emax_ref[...] = m

    @pl.when(i > 0)
    def _():
        emax_ref[...] = jnp.maximum(emax_ref[...], m)


def _edge_enc(eap, wp, b, eblk=8000):
    grid = (E // eblk,)
    return pl.pallas_call(
        _edge_enc_body,
        grid=grid,
        in_specs=[
            pl.BlockSpec((eblk, 8), lambda i: (i, 0)),
            pl.BlockSpec((8, HW), lambda i: (0, 0)),
            pl.BlockSpec((1, HW), lambda i: (0, 0)),
            pl.BlockSpec((8, HW), lambda i: (0, 0)),
            pl.BlockSpec((1, HW), lambda i: (0, 0)),
        ],
        out_specs=(
            pl.BlockSpec((eblk, HW), lambda i: (i, 0)),
            pl.BlockSpec((1, H), lambda i: (0, 0)),
        ),
        out_shape=(
            jax.ShapeDtypeStruct((E, HW), jnp.int32),
            jax.ShapeDtypeStruct((1, H), jnp.float32),
        ),
    )(eap, wp[:, :HW], b[:, :HW], wp[:, HW:], b[:, HW:])


def _post_mid_body(h_ref, s_ref, u_ref, w_ref, b_ref, g_ref, be_ref,
                   out_ref, omax_ref):
    x = h_ref[...] + u_ref[...] / (s_ref[...] + 1e-16)
    z = jnp.dot(x, w_ref[...], preferred_element_type=jnp.float32) + b_ref[...]
    mu = jnp.mean(z, axis=-1, keepdims=True)
    zc = z - mu
    var = jnp.mean(zc * zc, axis=-1, keepdims=True)
    inv = lax.rsqrt(var + 1e-05)
    y = jnp.maximum(zc * inv * g_ref[...] + be_ref[...], 0.0)
    out_ref[...] = y
    omax_ref[...] = jnp.max(y, axis=0, keepdims=True)


def _post_mid(h, s, u, w, b, g, be):
    return pl.pallas_call(
        _post_mid_body,
        out_shape=(
            jax.ShapeDtypeStruct((N, H), jnp.float32),
            jax.ShapeDtypeStruct((1, H), jnp.float32),
        ),
    )(h, s, u, w, b, g, be)


def _post_final_body(h_ref, s_ref, u_ref, w_ref, b_ref, g_ref, be_ref,
                     out_ref):
    x = h_ref[...] + u_ref[...] / (s_ref[...] + 1e-16)
    z = jnp.dot(x, w_ref[...], preferred_element_type=jnp.float32) + b_ref[...]
    mu = jnp.mean(z, axis=-1, keepdims=True)
    zc = z - mu
    var = jnp.mean(zc * zc, axis=-1, keepdims=True)
    out_ref[...] = zc * lax.rsqrt(var + 1e-05) * g_ref[...] + be_ref[...]


def _post_final(h, s, u, w, b, g, be):
    return pl.pallas_call(
        _post_final_body,
        out_shape=jax.ShapeDtypeStruct((N, H), jnp.float32),
    )(h, s, u, w, b, g, be)


def _sc_aggr_body(h_hbm, emb_hbm, src_hbm, dst_hbm, bnd_hbm,
                  s_out, u_out,
                  src_v, dst_v, rows_v, emb_v, out_v, bnd_v,
                  accum, sem_emb0, sem_emb1, sem_rows0, sem_rows1,
                  sem_sc0, sem_sc1, sem_idx):
    sem_emb = (sem_emb0, sem_emb1)
    sem_rows = (sem_rows0, sem_rows1)
    sem_sc = (sem_sc0, sem_sc1)
    c = lax.axis_index("c")
    t = lax.axis_index("s")
    t0 = t * NCH_T

    pltpu.sync_copy(bnd_hbm, bnd_v)

    def zrow(r, carry):
        for v in range(8):
            out_v[0, r, pl.ds(v * 16, 16)] = jnp.zeros((16,), jnp.float32)
        return carry

    lax.fori_loop(0, K, zrow, 0)
    for z in range(ROWS_T // K):
        pltpu.sync_copy(out_v.at[0], accum.at[pl.ds(t * ROWS_T + z * K, K)])
    plsc.subcore_barrier()

    def emb_slice(ci):
        return emb_hbm.at[pl.ds((t0 + ci) * K, K)]

    def src_row(ci):
        return src_v.at[lax.rem(lax.div(ci, SUP), 3), lax.rem(ci, SUP)]

    def dst_row(ci):
        return dst_v.at[lax.rem(lax.div(ci, SUP), 3), lax.rem(ci, SUP)]

    def load_super(sj):
        slot = lax.rem(sj, 3)
        pltpu.async_copy(src_hbm.at[t * NSUP_T + sj], src_v.at[slot], sem_idx)
        pltpu.async_copy(dst_hbm.at[t * NSUP_T + sj], dst_v.at[slot], sem_idx)

    def wait_super():
        pltpu.make_async_copy(src_hbm.at[t * NSUP_T], src_v.at[0],
                              sem_idx).wait()
        pltpu.make_async_copy(dst_hbm.at[t * NSUP_T], dst_v.at[0],
                              sem_idx).wait()

    def start_chunk(ci, b):
        pltpu.async_copy(emb_slice(ci), emb_v.at[b], sem_emb[b])
        pltpu.async_copy(h_hbm.at[src_row(ci)], rows_v.at[b], sem_rows[b])

    load_super(0)
    wait_super()
    start_chunk(0, 0)

    bnd8 = tuple(bnd_v[0, pl.ds(v * 16, 16)] for v in range(8))
    himask = jnp.full((16,), -65536, jnp.int32)

    def compute(ci, b):

        def msg(r, g):
            hlo = rows_v[b, r, pl.ds(16 * g, 16)]
            hhi = rows_v[b, r, pl.ds(64 + 16 * g, 16)]
            we = emb_v[b, r, pl.ds(16 * g, 16)]
            elo = lax.bitcast_convert_type(lax.shift_left(we, 16),
                                           jnp.float32)
            ehi = lax.bitcast_convert_type(we & himask, jnp.float32)
            m1 = jnp.maximum(hlo + elo, 0.0) + EPS
            m2 = jnp.maximum(hhi + ehi, 0.0) + EPS
            return m1, m2

        @pl.when(c == 0)
        def _():
            def row(r, carry):
                for g in range(4):
                    m1, m2 = msg(r, g)
                    out_v[b, r, pl.ds(16 * g, 16)] = jnp.exp(m1 - carry[g])
                    out_v[b, r, pl.ds(64 + 16 * g, 16)] = (
                        jnp.exp(m2 - carry[4 + g]))
                return carry

            lax.fori_loop(0, K, row, bnd8)

        @pl.when(c == 1)
        def _():
            def row(r, carry):
                for g in range(4):
                    m1, m2 = msg(r, g)
                    out_v[b, r, pl.ds(16 * g, 16)] = (
                        jnp.exp(m1 - carry[g]) * m1)
                    out_v[b, r, pl.ds(64 + 16 * g, 16)] = (
                        jnp.exp(m2 - carry[4 + g]) * m2)
                return carry

            lax.fori_loop(0, K, row, bnd8)

    def pair(g, carry):
        for b in range(2):
            ci = 2 * g + b
            if b == 0:
                @pl.when((lax.rem(g, SUP // 2) == 0)
                         & (lax.div(ci, SUP) + 1 < NSUP_T))
                def _():
                    load_super(lax.div(ci, SUP) + 1)

            @pl.when((lax.rem(ci + 1, SUP) == 0) & (ci + 1 < NCH_T))
            def _():
                wait_super()

            @pl.when(ci + 1 < NCH_T)
            def _():
                start_chunk(ci + 1, 1 - b)

            pltpu.make_async_copy(emb_slice(0), emb_v.at[b],
                                  sem_emb[b]).wait()
            pltpu.make_async_copy(h_hbm.at[src_v.at[0, 0]], rows_v.at[b],
                                  sem_rows[b]).wait()

            @pl.when(ci >= 2)
            def _():
                pltpu.make_async_copy(out_v.at[b], accum.at[dst_v.at[0, 0]],
                                      sem_sc[b]).wait()

            compute(ci, b)
            pltpu.async_copy(out_v.at[b], accum.at[dst_row(ci)], sem_sc[b],
                             add=True)
        return carry

    lax.fori_loop(0, NCH_T // 2, pair, 0)
    for b in range(2):
        pltpu.make_async_copy(out_v.at[b], accum.at[dst_v.at[0, 0]],
                              sem_sc[b]).wait()
    plsc.subcore_barrier()

    @pl.when(c == 0)
    def _():
        pltpu.sync_copy(accum.at[pl.ds(t * ROWS_T, ROWS_T)],
                        s_out.at[pl.ds(t * ROWS_T, ROWS_T)])

    @pl.when(c == 1)
    def _():
        pltpu.sync_copy(accum.at[pl.ds(t * ROWS_T, ROWS_T)],
                        u_out.at[pl.ds(t * ROWS_T, ROWS_T)])


_sc_aggr = functools.partial(
    pl.kernel,
    out_type=(
        jax.ShapeDtypeStruct((NP, H), jnp.float32),
        jax.ShapeDtypeStruct((NP, H), jnp.float32),
    ),
    mesh=plsc.VectorSubcoreMesh(core_axis_name="c", subcore_axis_name="s"),
    scratch_types=[
        pltpu.VMEM((3, SUP, K), jnp.int32),
        pltpu.VMEM((3, SUP, K), jnp.int32),
        pltpu.VMEM((2, K, H), jnp.float32),
        pltpu.VMEM((2, K, HW), jnp.int32),
        pltpu.VMEM((2, K, H), jnp.float32),
        pltpu.VMEM((1, H), jnp.float32),
        pltpu.VMEM_SHARED((NP, H), jnp.float32),
        pltpu.SemaphoreType.DMA,
        pltpu.SemaphoreType.DMA,
        pltpu.SemaphoreType.DMA,
        pltpu.SemaphoreType.DMA,
        pltpu.SemaphoreType.DMA,
        pltpu.SemaphoreType.DMA,
        pltpu.SemaphoreType.DMA,
    ],
)(_sc_aggr_body)


def kernel(x, edge_index, edge_attr, W_node, b_node, W_edge, b_edge,
           Wc0, bc0, Wc1, bc1, g0, be0, g1, be1):
    xp = jnp.pad(x, ((0, 0), (0, 5)))
    Wn = jnp.pad(W_node, ((0, 5), (0, 0)))
    eap = jnp.pad(edge_attr, ((0, 0), (0, 1)))
    We = jnp.pad(W_edge, ((0, 1), (0, 0)))

    h, hmax = _node_enc(xp, Wn, b_node.reshape(1, H))
    emb, emax = _edge_enc(eap, We, b_edge.reshape(1, H))

    src3d = edge_index[0].reshape(NS * NSUP_T, SUP, K)
    dst3d = edge_index[1].reshape(NS * NSUP_T, SUP, K)

    bnd0 = jnp.maximum(hmax + emax, 0.0) + EPS
    s0, u0 = _sc_aggr(h, emb, src3d, dst3d, bnd0)
    h2, h2max = _post_mid(h, s0[:N], u0[:N], Wc0, bc0.reshape(1, H),
                          g0.reshape(1, H), be0.reshape(1, H))

    bnd1 = jnp.maximum(h2max + emax, 0.0) + EPS
    s1, u1 = _sc_aggr(h2, emb, src3d, dst3d, bnd1)
    return _post_final(h2, s1[:N], u1[:N], Wc1, bc1.reshape(1, H),
                       g1.reshape(1, H), be1.reshape(1, H))
